# baseline (speedup 1.0000x reference)
import functools

import jax
import jax.numpy as jnp
from jax import lax
from jax.experimental import pallas as pl
from jax.experimental.pallas import tpu as pltpu
from jax.experimental.pallas import tpu_sc as plsc

_B, _S, _H = 4, 8192, 768
_NW = 32
_SPW = _S // _NW
_CS = 8
_CHUNK = _CS * _H
_NCHUNK = _SPW // _CS
_NV = _CHUNK // 16


def _sc_body(x_hbm, pe_hbm, out_hbm, xbuf, pebuf):
    wid = lax.axis_index("s") * 2 + lax.axis_index("c")
    s0 = wid * _SPW

    def chunk_body(c, carry):
        s_off = s0 + c * _CS
        pe_off = s_off * _H
        pltpu.sync_copy(pe_hbm.at[pl.ds(pe_off, _CHUNK)], pebuf)
        for b in range(_B):
            x_off = (b * _S + s_off) * _H
            pltpu.sync_copy(x_hbm.at[pl.ds(x_off, _CHUNK)], xbuf.at[b])

        def add_body(j, carry2):
            sl = pl.ds(j * 16, 16)
            v = pebuf[sl]
            for b in range(_B):
                xbuf[b, sl] = xbuf[b, sl] + v
            return carry2

        lax.fori_loop(0, _NV, add_body, 0)

        for b in range(_B):
            x_off = (b * _S + s_off) * _H
            pltpu.sync_copy(xbuf.at[b], out_hbm.at[pl.ds(x_off, _CHUNK)])
        return carry

    lax.fori_loop(0, _NCHUNK, chunk_body, 0)


@jax.jit
def kernel(x_tok, pos_emb):
    x_flat = x_tok.reshape(-1)
    pe_flat = pos_emb.reshape(-1)
    out = pl.kernel(
        _sc_body,
        out_type=jax.ShapeDtypeStruct((_B * _S * _H,), jnp.float32),
        mesh=plsc.VectorSubcoreMesh(core_axis_name="c", subcore_axis_name="s"),
        scratch_types=[
            pltpu.VMEM((_B, _CHUNK), jnp.float32),
            pltpu.VMEM((_CHUNK,), jnp.float32),
        ],
    )(x_flat, pe_flat)
    return out.reshape(_B, _S, _H)


# speedup vs baseline: 1.4473x; 1.4473x over previous
import jax
import jax.numpy as jnp
from jax import lax
from jax.experimental import pallas as pl
from jax.experimental.pallas import tpu as pltpu
from jax.experimental.pallas import tpu_sc as plsc

_B, _S, _H = 4, 8192, 768
_NW = 32
_SPW = _S // _NW
_CS = 8
_CHUNK = _CS * _H
_NCHUNK = _SPW // _CS
_NV = _CHUNK // 16


def _sc_body(x_hbm, pe_hbm, out_hbm, xin, pein, obuf, in0, in1, out0, out1):
    wid = lax.axis_index("s") * 2 + lax.axis_index("c")
    s0 = wid * _SPW
    in_sems = (in0, in1)
    out_sems = (out0, out1)

    def issue_in(c, nb):
        s_off = s0 + c * _CS
        pltpu.async_copy(
            pe_hbm.at[pl.ds(s_off * _H, _CHUNK)], pein.at[nb], in_sems[nb])
        for bb in range(_B):
            pltpu.async_copy(
                x_hbm.at[pl.ds((bb * _S + s_off) * _H, _CHUNK)],
                xin.at[nb, bb], in_sems[nb])

    def wait_in(nb):
        pltpu.make_async_copy(
            pe_hbm.at[pl.ds(0, _CHUNK)], pein.at[nb], in_sems[nb]).wait()
        for bb in range(_B):
            pltpu.make_async_copy(
                x_hbm.at[pl.ds(0, _CHUNK)], xin.at[nb, bb], in_sems[nb]).wait()

    def issue_out(c, nb):
        s_off = s0 + c * _CS
        for bb in range(_B):
            pltpu.async_copy(
                obuf.at[nb, bb],
                out_hbm.at[pl.ds((bb * _S + s_off) * _H, _CHUNK)],
                out_sems[nb])

    def wait_out(nb):
        for bb in range(_B):
            pltpu.make_async_copy(
                obuf.at[nb, bb], out_hbm.at[pl.ds(0, _CHUNK)],
                out_sems[nb]).wait()

    issue_in(0, 0)
    issue_in(1, 1)

    def group(g, carry):
        for nb in range(2):
            c = g * 2 + nb
            wait_in(nb)

            @pl.when(g >= 1)
            def _():
                wait_out(nb)

            @plsc.parallel_loop(0, _NV, unroll=8)
            def _add(j):
                sl = pl.ds(j * 16, 16)
                pev = pein[nb, sl]
                for bb in range(_B):
                    obuf[nb, bb, sl] = xin[nb, bb, sl] + pev

            issue_out(c, nb)

            @pl.when(c + 2 < _NCHUNK)
            def _():
                issue_in(c + 2, nb)
        return carry

    lax.fori_loop(0, _NCHUNK // 2, group, 0)
    wait_out(0)
    wait_out(1)


@jax.jit
def kernel(x_tok, pos_emb):
    x_flat = x_tok.reshape(-1)
    pe_flat = pos_emb.reshape(-1)
    out = pl.kernel(
        _sc_body,
        out_type=jax.ShapeDtypeStruct((_B * _S * _H,), jnp.float32),
        mesh=plsc.VectorSubcoreMesh(core_axis_name="c", subcore_axis_name="s"),
        scratch_types=[
            pltpu.VMEM((2, _B, _CHUNK), jnp.float32),
            pltpu.VMEM((2, _CHUNK), jnp.float32),
            pltpu.VMEM((2, _B, _CHUNK), jnp.float32),
            pltpu.SemaphoreType.DMA,
            pltpu.SemaphoreType.DMA,
            pltpu.SemaphoreType.DMA,
            pltpu.SemaphoreType.DMA,
        ],
    )(x_flat, pe_flat)
    return out.reshape(_B, _S, _H)


# speedup vs baseline: 4.9776x; 3.4392x over previous
import jax
import jax.numpy as jnp
from jax import lax
from jax.experimental import pallas as pl
from jax.experimental.pallas import tpu as pltpu
from jax.experimental.pallas import tpu_sc as plsc

_B, _S, _H = 4, 8192, 768
_NW = 32
_SPW = _S // _NW
_CS = 8
_NCHUNK = _SPW // _CS
_NVROW = _H // 16


def _sc_body(x_hbm, pe_hbm, out_hbm, xin, pein, obuf, in0, in1, out0, out1):
    wid = lax.axis_index("s") * 2 + lax.axis_index("c")
    s0 = wid * _SPW
    in_sems = (in0, in1)
    out_sems = (out0, out1)

    def issue_in(c, nb):
        s_off = s0 + c * _CS
        pltpu.async_copy(
            pe_hbm.at[pl.ds(s_off, _CS), :], pein.at[nb], in_sems[nb])
        for bb in range(_B):
            pltpu.async_copy(
                x_hbm.at[bb, pl.ds(s_off, _CS), :], xin.at[nb, bb],
                in_sems[nb])

    def wait_in(nb):
        pltpu.make_async_copy(
            pe_hbm.at[pl.ds(0, _CS), :], pein.at[nb], in_sems[nb]).wait()
        for bb in range(_B):
            pltpu.make_async_copy(
                x_hbm.at[bb, pl.ds(0, _CS), :], xin.at[nb, bb],
                in_sems[nb]).wait()

    def issue_out(c, nb):
        s_off = s0 + c * _CS
        for bb in range(_B):
            pltpu.async_copy(
                obuf.at[nb, bb], out_hbm.at[bb, pl.ds(s_off, _CS), :],
                out_sems[nb])

    def wait_out(nb):
        for bb in range(_B):
            pltpu.make_async_copy(
                obuf.at[nb, bb], out_hbm.at[bb, pl.ds(0, _CS), :],
                out_sems[nb]).wait()

    issue_in(0, 0)
    issue_in(1, 1)

    def group(g, carry):
        for nb in range(2):
            c = g * 2 + nb
            wait_in(nb)

            @pl.when(g >= 1)
            def _():
                wait_out(nb)

            for r in range(_CS):
                @plsc.parallel_loop(0, _NVROW, unroll=8)
                def _add(j):
                    sl = pl.ds(j * 16, 16)
                    pev = pein[nb, r, sl]
                    for bb in range(_B):
                        obuf[nb, bb, r, sl] = xin[nb, bb, r, sl] + pev

            issue_out(c, nb)

            @pl.when(c + 2 < _NCHUNK)
            def _():
                issue_in(c + 2, nb)
        return carry

    lax.fori_loop(0, _NCHUNK // 2, group, 0)
    wait_out(0)
    wait_out(1)


@jax.jit
def kernel(x_tok, pos_emb):
    return pl.kernel(
        _sc_body,
        out_type=jax.ShapeDtypeStruct((_B, _S, _H), jnp.float32),
        mesh=plsc.VectorSubcoreMesh(core_axis_name="c", subcore_axis_name="s"),
        scratch_types=[
            pltpu.VMEM((2, _B, _CS, _H), jnp.float32),
            pltpu.VMEM((2, _CS, _H), jnp.float32),
            pltpu.VMEM((2, _B, _CS, _H), jnp.float32),
            pltpu.SemaphoreType.DMA,
            pltpu.SemaphoreType.DMA,
            pltpu.SemaphoreType.DMA,
            pltpu.SemaphoreType.DMA,
        ],
    )(x_tok, pos_emb)


# speedup vs baseline: 4.9780x; 1.0001x over previous
import jax
import jax.numpy as jnp
from jax import lax
from jax.experimental import pallas as pl
from jax.experimental.pallas import tpu as pltpu
from jax.experimental.pallas import tpu_sc as plsc

_B, _S, _H = 4, 8192, 768
_NW = 32
_SPW = _S // _NW
_CS = 8
_NCHUNK = _SPW // _CS
_NVROW = _H // 16


def _sc_body(x_hbm, pe_hbm, out_hbm, xin, pein, obuf, in0, in1, out0, out1):
    wid = lax.axis_index("s") * 2 + lax.axis_index("c")
    s0 = wid * _SPW
    in_sems = (in0, in1)
    out_sems = (out0, out1)

    def issue_in(c, nb):
        s_off = s0 + c * _CS
        pltpu.async_copy(
            pe_hbm.at[pl.ds(s_off, _CS), :], pein.at[nb], in_sems[nb])
        for bb in range(_B):
            pltpu.async_copy(
                x_hbm.at[bb, pl.ds(s_off, _CS), :], xin.at[nb, bb],
                in_sems[nb])

    def wait_in(nb):
        pltpu.make_async_copy(
            pe_hbm.at[pl.ds(0, _CS), :], pein.at[nb], in_sems[nb]).wait()
        for bb in range(_B):
            pltpu.make_async_copy(
                x_hbm.at[bb, pl.ds(0, _CS), :], xin.at[nb, bb],
                in_sems[nb]).wait()

    def issue_out(c, nb):
        s_off = s0 + c * _CS
        for bb in range(_B):
            pltpu.async_copy(
                obuf.at[nb, bb], out_hbm.at[bb, pl.ds(s_off, _CS), :],
                out_sems[nb])

    def wait_out(nb):
        for bb in range(_B):
            pltpu.make_async_copy(
                obuf.at[nb, bb], out_hbm.at[bb, pl.ds(0, _CS), :],
                out_sems[nb]).wait()

    issue_in(0, 0)
    issue_in(1, 1)

    def group(g, carry):
        for nb in range(2):
            c = g * 2 + nb
            wait_in(nb)

            @pl.when(g >= 1)
            def _():
                wait_out(nb)

            @plsc.parallel_loop(0, _NVROW, unroll=2)
            def _add(j):
                sl = pl.ds(j * 16, 16)
                for r in range(_CS):
                    pev = pein[nb, r, sl]
                    for bb in range(_B):
                        obuf[nb, bb, r, sl] = xin[nb, bb, r, sl] + pev

            @pl.when(c + 2 < _NCHUNK)
            def _():
                issue_in(c + 2, nb)

            issue_out(c, nb)
        return carry

    lax.fori_loop(0, _NCHUNK // 2, group, 0)
    wait_out(0)
    wait_out(1)


@jax.jit
def kernel(x_tok, pos_emb):
    return pl.kernel(
        _sc_body,
        out_type=jax.ShapeDtypeStruct((_B, _S, _H), jnp.float32),
        mesh=plsc.VectorSubcoreMesh(core_axis_name="c", subcore_axis_name="s"),
        scratch_types=[
            pltpu.VMEM((2, _B, _CS, _H), jnp.float32),
            pltpu.VMEM((2, _CS, _H), jnp.float32),
            pltpu.VMEM((2, _B, _CS, _H), jnp.float32),
            pltpu.SemaphoreType.DMA,
            pltpu.SemaphoreType.DMA,
            pltpu.SemaphoreType.DMA,
            pltpu.SemaphoreType.DMA,
        ],
    )(x_tok, pos_emb)
